# baseline (speedup 1.0000x reference)
import functools

import jax
import jax.numpy as jnp
from jax import lax
from jax.experimental import pallas as pl
from jax.experimental.pallas import tpu as pltpu
from jax.experimental.pallas import tpu_sc as plsc

D = 32
B = 16384
L = 50
S = L + 1
NC = 2
NS = 16
NW = NC * NS
BPW = B // NW
CH = 32
ROWS = CH * S
NCHUNK = BPW // CH
G = 96
NG = ROWS // G


def _embed_bag_body(idx_hbm, table_hbm, out_hbm, idx_v, buf_v, out_v, sem):
    wid = lax.axis_index("s") * NC + lax.axis_index("c")
    base_b = wid * BPW

    def chunk_body(c, carry):
        flat_base = pl.multiple_of((base_b + c * CH) * S, 8)
        pltpu.sync_copy(idx_hbm.at[pl.ds(flat_base, ROWS)],
                        idx_v.at[pl.ds(0, ROWS)])

        copies = []
        for j in range(NG):
            copies.append(pltpu.async_copy(
                table_hbm.at[idx_v.at[pl.ds(j * G, G)]],
                buf_v.at[pl.ds(j * G, G)],
                sem))
        for cp in copies:
            cp.wait()

        def batch_body(b, bcarry):
            r0 = b * S
            acc0 = buf_v[r0, pl.ds(0, 16)]
            acc1 = buf_v[r0, pl.ds(16, 16)]
            for l in range(1, L):
                acc0 = acc0 + buf_v[r0 + l, pl.ds(0, 16)]
                acc1 = acc1 + buf_v[r0 + l, pl.ds(16, 16)]
            lnv = idx_v[pl.ds(r0 + L, 16)]
            lf = jnp.maximum(lnv[0], 1).astype(jnp.float32)
            out_v[b, pl.ds(0, 16)] = acc0 / lf
            out_v[b, pl.ds(16, 16)] = acc1 / lf
            return bcarry

        lax.fori_loop(0, CH, batch_body, 0)

        out_base = pl.multiple_of(base_b + c * CH, 8)
        pltpu.sync_copy(out_v, out_hbm.at[pl.ds(out_base, CH)])
        return carry

    lax.fori_loop(0, NCHUNK, chunk_body, 0)


@jax.jit
def _embed_bag(idx_stream, table):
    mesh = plsc.VectorSubcoreMesh(core_axis_name="c", subcore_axis_name="s")
    return pl.kernel(
        _embed_bag_body,
        out_type=jax.ShapeDtypeStruct((B, D), jnp.float32),
        mesh=mesh,
        compiler_params=pltpu.CompilerParams(use_tc_tiling_on_sc=False),
        scratch_types=[
            pltpu.VMEM((ROWS + 16,), jnp.int32),
            pltpu.VMEM((ROWS, D), jnp.float32),
            pltpu.VMEM((CH, D), jnp.float32),
            pltpu.SemaphoreType.DMA,
        ],
    )(idx_stream, table)


def kernel(kw_indices, kw_lengths, embedding_weight):
    stream = jnp.concatenate(
        [kw_indices.astype(jnp.int32), kw_lengths.astype(jnp.int32)], axis=1)
    return _embed_bag(stream.reshape(-1), embedding_weight)
